# Initial kernel scaffold; baseline (speedup 1.0000x reference)
import jax
import jax.numpy as jnp
from jax.experimental import pallas as pl


def kernel(cosine, label):
    raise NotImplementedError("write your pallas kernel here")


# speedup vs baseline: 1.0262x; 1.0262x over previous
import functools

import jax
import jax.numpy as jnp
from jax.experimental import pallas as pl

_S = 64.0
_M = 0.4

_BLOCK_COLS = 2048


def _cosface_block(cosine_ref, label_ref, out_ref):
    j = pl.program_id(0)
    cols = j * _BLOCK_COLS + jax.lax.broadcasted_iota(
        jnp.int32, (cosine_ref.shape[0], _BLOCK_COLS), 1)
    lbl = label_ref[...]
    margin = jnp.where(cols == lbl, -_S * _M, 0.0).astype(cosine_ref.dtype)
    out_ref[...] = cosine_ref[...] * _S + margin


@functools.partial(jax.jit, static_argnames=())
def kernel(cosine, label):
    rows, n_cols = cosine.shape
    grid = (pl.cdiv(n_cols, _BLOCK_COLS),)
    lbl2d = label.reshape(rows, 1)
    return pl.pallas_call(
        _cosface_block,
        grid=grid,
        in_specs=[
            pl.BlockSpec((rows, _BLOCK_COLS), lambda j: (0, j)),
            pl.BlockSpec((rows, 1), lambda j: (0, 0)),
        ],
        out_specs=pl.BlockSpec((rows, _BLOCK_COLS), lambda j: (0, j)),
        out_shape=jax.ShapeDtypeStruct((rows, n_cols), cosine.dtype),
    )(cosine, lbl2d)
